# baseline (speedup 1.0000x reference)
import functools
import jax
import jax.numpy as jnp
from jax.experimental import pallas as pl
from jax.experimental.pallas import tpu as pltpu

NUM_EXPERTS = 8
TOP_K = 2


def _moe_body(x_ref, semb_ref, wr_ref, br_ref, w1_ref, b1_ref, w2_ref, b2_ref,
              out_ref, scale_ref):
    e = pl.program_id(0)
    x = x_ref[...]

    @pl.when(e == 0)
    def _router():
        xr = x + semb_ref[...]
        logits = jnp.dot(xr, wr_ref[...].T,
                         preferred_element_type=jnp.float32) + br_ref[...]
        S = logits.shape[0]
        ecol = jax.lax.broadcasted_iota(jnp.int32, (S, NUM_EXPERTS), 1)
        big = jnp.int32(NUM_EXPERTS)
        l0 = jnp.max(logits, axis=-1, keepdims=True)
        i0 = jnp.min(jnp.where(logits == l0, ecol, big), axis=-1, keepdims=True)
        masked = jnp.where(ecol == i0, -jnp.inf, logits)
        l1 = jnp.max(masked, axis=-1, keepdims=True)
        i1 = jnp.min(jnp.where(masked == l1, ecol, big), axis=-1, keepdims=True)
        w0 = 1.0 / (1.0 + jnp.exp(l1 - l0))
        w1 = 1.0 - w0
        scale_ref[...] = (jnp.where(ecol == i0, w0, 0.0)
                          + jnp.where(ecol == i1, w1, 0.0))

    h = jnp.dot(x, w1_ref[0].T, preferred_element_type=jnp.float32) + b1_ref[0]
    h = jax.nn.gelu(h, approximate=True)
    y = jnp.dot(h, w2_ref[0].T, preferred_element_type=jnp.float32) + b2_ref[0]
    S = x.shape[0]
    ecol = jax.lax.broadcasted_iota(jnp.int32, (S, NUM_EXPERTS), 1)
    sc = jnp.sum(jnp.where(ecol == e, scale_ref[...], 0.0), axis=-1,
                 keepdims=True)
    contrib = y * sc

    @pl.when(e == 0)
    def _init():
        out_ref[...] = contrib

    @pl.when(e > 0)
    def _acc():
        out_ref[...] += contrib


def kernel(x, scale_emb, Wr, br, W1, b1, W2, b2, scale_idx):
    B, S, D = x.shape
    E, H, _ = W1.shape
    xs = x.reshape(B * S, D)
    semb = jax.lax.dynamic_slice_in_dim(scale_emb, scale_idx, 1, axis=0)

    out = pl.pallas_call(
        _moe_body,
        grid=(E,),
        in_specs=[
            pl.BlockSpec((B * S, D), lambda e: (0, 0)),
            pl.BlockSpec((1, D), lambda e: (0, 0)),
            pl.BlockSpec((E, D), lambda e: (0, 0)),
            pl.BlockSpec((1, E), lambda e: (0, 0)),
            pl.BlockSpec((1, H, D), lambda e: (e, 0, 0)),
            pl.BlockSpec((1, 1, H), lambda e: (e, 0, 0)),
            pl.BlockSpec((1, D, H), lambda e: (e, 0, 0)),
            pl.BlockSpec((1, 1, D), lambda e: (e, 0, 0)),
        ],
        out_specs=pl.BlockSpec((B * S, D), lambda e: (0, 0)),
        out_shape=jax.ShapeDtypeStruct((B * S, D), jnp.float32),
        scratch_shapes=[pltpu.VMEM((B * S, E), jnp.float32)],
    )(xs, semb, Wr, br.reshape(1, E), W1, b1.reshape(E, 1, H),
      W2, b2.reshape(E, 1, D))
    return out.reshape(B, S, D)
